# Initial kernel scaffold; baseline (speedup 1.0000x reference)
import jax
import jax.numpy as jnp
from jax.experimental import pallas as pl


def kernel(x, enc_params, dec_params):
    raise NotImplementedError("write your pallas kernel here")


# speedup vs baseline: 2.2413x; 2.2413x over previous
import jax
import jax.numpy as jnp
from jax import lax
from jax.experimental import pallas as pl
from jax.experimental.pallas import tpu as pltpu

_DN = ('NCHW', 'OIHW', 'NCHW')


def _conv(x, w, b, stride=1):
    y = lax.conv_general_dilated(x, w, (stride, stride), ((1, 1), (1, 1)),
                                 dimension_numbers=_DN)
    return y + b[None, :, None, None]


def _deconv(x, w, b):
    y = lax.conv_general_dilated(x, w, (1, 1), ((1, 2), (1, 2)),
                                 lhs_dilation=(2, 2), dimension_numbers=_DN)
    return y + b[None, :, None, None]


def _gdn(x, beta, gamma, inverse=False):
    norm = jnp.sqrt(jnp.einsum('bihw,oi->bohw', x * x, gamma)
                    + beta[None, :, None, None])
    return x * norm if inverse else x / norm


def _resblock(x, w1, b1, w2, b2):
    h = jax.nn.relu(_conv(x, w1, b1, 1))
    return _conv(h, w2, b2, 1) + x


_LANES = 128
_BINS = 256
_CHUNK = 64


def _sig_ent_kernel(d_ref, rec_ref, ent_ref, idx_ref):
    d = d_ref[0]
    rec = jax.nn.sigmoid(d)
    rec_ref[0] = rec
    mn = jnp.min(rec)
    mx = jnp.max(rec)
    scale = jnp.where(mx > mn, 256.0 / (mx - mn), 0.0)
    idx_ref[...] = jnp.clip(jnp.floor((rec - mn) * scale), 0.0, 255.0
                            ).astype(jnp.int32)

    rows = d.shape[0]
    bins = lax.broadcasted_iota(jnp.int32, (_BINS, _CHUNK, _LANES), 0)

    def body(i, hist):
        chunk = idx_ref[pl.ds(i * _CHUNK, _CHUNK), :]
        eq = jnp.where(bins == chunk[None, :, :], 1.0, 0.0)
        return hist + jnp.sum(eq, axis=1)

    hist = lax.fori_loop(0, rows // _CHUNK, body,
                         jnp.zeros((_BINS, _LANES), jnp.float32))
    histv = jnp.sum(hist, axis=1)
    total = jnp.sum(histv)
    p = histv / total
    ent = -jnp.sum(p * jnp.log2(p + 1e-6))
    ent_ref[0] = jnp.full((8, _LANES), ent, jnp.float32)


def _sigmoid_entropy(d):
    n, c, h, w = d.shape
    rows = (c * h * w) // _LANES
    d2 = d.reshape(n, rows, _LANES)
    rec2, ent = pl.pallas_call(
        _sig_ent_kernel,
        grid=(n,),
        in_specs=[pl.BlockSpec((1, rows, _LANES), lambda i: (i, 0, 0))],
        out_specs=[pl.BlockSpec((1, rows, _LANES), lambda i: (i, 0, 0)),
                   pl.BlockSpec((1, 8, _LANES), lambda i: (i, 0, 0))],
        out_shape=[jax.ShapeDtypeStruct((n, rows, _LANES), jnp.float32),
                   jax.ShapeDtypeStruct((n, 8, _LANES), jnp.float32)],
        scratch_shapes=[pltpu.VMEM((rows, _LANES), jnp.int32)],
        compiler_params=pltpu.CompilerParams(
            dimension_semantics=("parallel",)),
    )(d2)
    return rec2.reshape(n, c, h, w), ent[:, 0, 0]


def kernel(x, enc_params, dec_params):
    ep, dp = enc_params, dec_params
    h = _conv(x, ep['w0'], ep['b0'], 2)
    h = _gdn(h, ep['beta0'], ep['gamma0'])
    h = _conv(h, ep['w1'], ep['b1'], 2)
    h = _gdn(h, ep['beta1'], ep['gamma1'])
    h = _conv(h, ep['w2'], ep['b2'], 2)
    h = _gdn(h, ep['beta2'], ep['gamma2'])
    h = _conv(h, ep['w3'], ep['b3'], 2)
    latent = _resblock(h, ep['rw1'], ep['rb1'], ep['rw2'], ep['rb2'])

    b = latent + lax.stop_gradient(jnp.sign(latent) - latent)

    d = _deconv(b, dp['w0'], dp['b0'])
    d = _gdn(d, dp['beta0'], dp['gamma0'], inverse=True)
    d = _deconv(d, dp['w1'], dp['b1'])
    d = _gdn(d, dp['beta1'], dp['gamma1'], inverse=True)
    d = _deconv(d, dp['w2'], dp['b2'])
    d = _gdn(d, dp['beta2'], dp['gamma2'], inverse=True)
    d = _deconv(d, dp['w3'], dp['b3'])
    d = _resblock(d, dp['rw1'], dp['rb1'], dp['rw2'], dp['rb2'])

    reconstructed, ent = _sigmoid_entropy(d)
    entropy = jnp.mean(ent)
    return reconstructed, latent, entropy
